# baseline (speedup 1.0000x reference)
import functools

import jax
import jax.numpy as jnp
from jax import lax
from jax.experimental import pallas as pl
from jax.experimental.pallas import tpu as pltpu
from jax.experimental.pallas import tpu_sc as plsc

_V = 100000
_E = 64
_H = 256
_B = 16384
_L = 200
_NC = 2
_NS = 16
_NW = _NC * _NS
_BPW = _B // _NW
_CB = 32
_NBLK = _BPW // _CB
_L0 = 128
_L1 = _L - _L0
_LANES = 16
_R = 8


def _pool_body(tokens_hbm, table_hbm, pooled_hbm, idx_v,
               ga, gb, gc, gd, ge, gf, gg, gh, obuf,
               sem_i, sa, sb, sc, sd, se, sf, sg_, sh, so0, so1):
    c = lax.axis_index("c")
    s = lax.axis_index("s")
    wid = s * _NC + c
    base = wid * _BPW
    inv_l = jnp.float32(1.0 / _L)
    bufs = ((ga, sa), (gb, sb), (gc, sc), (gd, sd),
            (ge, se), (gf, sf), (gg, sg_), (gh, sh))
    osems = (so0, so1)

    def fire_idx(blk, islot):
        pltpu.async_copy(
            tokens_hbm.at[pl.ds((base + blk * _CB) * _L, _CB * _L)],
            idx_v.at[islot], sem_i)

    def wait_idx():
        pltpu.make_async_copy(tokens_hbm.at[pl.ds(base * _L, _CB * _L)],
                              idx_v.at[0], sem_i).wait()

    def fire_gath(islot, rib, gs):
        g, sem = bufs[gs]
        pltpu.async_copy(
            table_hbm.at[idx_v.at[islot, pl.ds(rib * _L, _L0)]],
            g.at[pl.ds(0, _L0)], sem)
        pltpu.async_copy(
            table_hbm.at[idx_v.at[islot, pl.ds(rib * _L + _L0, _L1)]],
            g.at[pl.ds(_L0, _L1)], sem)

    def wait_gath(gs):
        g, sem = bufs[gs]
        pltpu.make_async_copy(table_hbm.at[idx_v.at[0, pl.ds(0, _L0)]],
                              g.at[pl.ds(0, _L0)], sem).wait()
        pltpu.make_async_copy(table_hbm.at[idx_v.at[0, pl.ds(_L0, _L1)]],
                              g.at[pl.ds(_L0, _L1)], sem).wait()

    def reduce_row(gs, oslot, rib):
        g, _ = bufs[gs]
        mhi = jnp.int32(-65536)

        def red(i, acc):
            w0 = g[i, pl.ds(0, _LANES)]
            w1 = g[i, pl.ds(_LANES, _LANES)]
            e0 = lax.bitcast_convert_type(w0 << 16, jnp.float32)
            o0 = lax.bitcast_convert_type(w0 & mhi, jnp.float32)
            e1 = lax.bitcast_convert_type(w1 << 16, jnp.float32)
            o1 = lax.bitcast_convert_type(w1 & mhi, jnp.float32)
            return (acc[0] + e0, acc[1] + o0, acc[2] + e1, acc[3] + o1)

        z = jnp.zeros((_LANES,), jnp.float32)
        acc = lax.fori_loop(0, _L, red, (z, z, z, z), unroll=8)
        for j in range(4):
            obuf[oslot, pl.ds(rib * _E + j * _LANES, _LANES)] = acc[j] * inv_l

    def fire_out(blk, oslot):
        pltpu.async_copy(
            obuf.at[oslot],
            pooled_hbm.at[pl.ds((base + blk * _CB) * _E, _CB * _E)],
            osems[oslot])

    def wait_out(oslot):
        pltpu.make_async_copy(obuf.at[oslot],
                              pooled_hbm.at[pl.ds(base * _E, _CB * _E)],
                              osems[oslot]).wait()

    pltpu.sync_copy(tokens_hbm.at[pl.ds(base * _L, _CB * _L)], idx_v.at[0])
    for k in range(_R - 1):
        fire_gath(0, k, k)

    @pl.loop(0, _NBLK, step=2)
    def _(blk0):
        for di in range(2):
            blk = blk0 + di
            islot = di
            oslot = di
            nislot = 1 - di
            has_next = blk + 1 < _NBLK

            @pl.when(has_next)
            def _():
                fire_idx(blk + 1, nislot)

            @pl.when(blk >= 2)
            def _():
                wait_out(oslot)

            @pl.loop(0, (_CB - _R) // _R)
            def _(p):
                r = _R * p
                for k in range(_R):
                    fire_gath(islot, r + k + _R - 1, (k + _R - 1) % _R)
                    wait_gath(k)
                    reduce_row(k, oslot, r + k)

            rt = _CB - _R
            fire_gath(islot, _CB - 1, _R - 1)
            wait_gath(0)
            reduce_row(0, oslot, rt)

            @pl.when(has_next)
            def _():
                wait_idx()
                fire_gath(nislot, 0, 0)

            wait_gath(1)
            reduce_row(1, oslot, rt + 1)
            for k in range(2, _R):
                @pl.when(has_next)
                def _(k=k):
                    fire_gath(nislot, k - 1, k - 1)

                wait_gath(k)
                reduce_row(k, oslot, rt + k)
            fire_out(blk, oslot)

    wait_out(0)
    wait_out(1)


@jax.jit
def _pool_sc(tokens, table):
    mesh = plsc.VectorSubcoreMesh(core_axis_name="c", subcore_axis_name="s")
    f = pl.kernel(
        _pool_body,
        out_type=jax.ShapeDtypeStruct((_B * _E,), jnp.float32),
        mesh=mesh,
        scratch_types=(
            [pltpu.VMEM((2, _CB * _L), jnp.int32)]
            + [pltpu.VMEM((_L, 32), jnp.int32) for _ in range(_R)]
            + [pltpu.VMEM((2, _CB * _E), jnp.float32)]
            + [pltpu.SemaphoreType.DMA] * (_R + 3)
        ),
        compiler_params=pltpu.CompilerParams(use_tc_tiling_on_sc=False),
    )
    return f(tokens, table)


_BB = 512


def _mlp_body(p_ref, w1_ref, b1_ref, w2_ref, b2_ref, o_ref):
    p = p_ref[...]
    h = lax.dot_general(p, w1_ref[...],
                        (((1,), (1,)), ((), ())),
                        preferred_element_type=jnp.float32)
    h = jnp.maximum(h + b1_ref[...], 0.0)
    o = jnp.sum(h * w2_ref[...], axis=1) + b2_ref[0]
    o_ref[...] = jax.nn.sigmoid(o)


@jax.jit
def _mlp_tc(pooled, W1, b1, W2, b2):
    grid = _B // _BB
    return pl.pallas_call(
        _mlp_body,
        grid=(grid,),
        in_specs=[
            pl.BlockSpec((_BB, _E), lambda i: (i, 0)),
            pl.BlockSpec((_H, _E), lambda i: (0, 0)),
            pl.BlockSpec((_H,), lambda i: (0,)),
            pl.BlockSpec((1, _H), lambda i: (0, 0)),
            pl.BlockSpec((1,), lambda i: (0,)),
        ],
        out_specs=pl.BlockSpec((_BB,), lambda i: (i,)),
        out_shape=jax.ShapeDtypeStruct((_B,), jnp.float32),
    )(pooled, W1, b1, W2, b2)


_PERM = (tuple(range(0, 32, 2)) + tuple(range(1, 32, 2))
         + tuple(range(32, 64, 2)) + tuple(range(33, 64, 2)))


def kernel(structure_tokens, table, W1, b1, W2, b2):
    tokens = structure_tokens.astype(jnp.int32).reshape(_B * _L)
    tbl = lax.bitcast_convert_type(
        table.astype(jnp.bfloat16).reshape(_V, _E // 2, 2), jnp.int32)
    pooled = _pool_sc(tokens, tbl).reshape(_B, _E)
    return _mlp_tc(pooled, W1[:, jnp.array(_PERM)], b1, W2, b2)


# speedup vs baseline: 1.3130x; 1.3130x over previous
import functools

import jax
import jax.numpy as jnp
from jax import lax
from jax.experimental import pallas as pl
from jax.experimental.pallas import tpu as pltpu
from jax.experimental.pallas import tpu_sc as plsc

_V = 100000
_E = 64
_H = 256
_B = 16384
_L = 200
_NC = 2
_NS = 16
_NW = _NC * _NS
_BPW = _B // _NW
_CB = 32
_NBLK = _BPW // _CB
_L0 = 128
_L1 = _L - _L0
_LANES = 16
_R = 8


def _pool_body(tokens_hbm, table_hbm, pooled_hbm, idx_v,
               ga, gb, gc, gd, ge, gf, gg, gh, obuf,
               sem_i, sa, sb, sc, sd, se, sf, sg_, sh, so0, so1):
    c = lax.axis_index("c")
    s = lax.axis_index("s")
    wid = s * _NC + c
    base = wid * _BPW
    inv_l = jnp.float32(1.0 / _L)
    bufs = ((ga, sa), (gb, sb), (gc, sc), (gd, sd),
            (ge, se), (gf, sf), (gg, sg_), (gh, sh))
    osems = (so0, so1)

    def fire_idx(blk, islot):
        pltpu.async_copy(
            tokens_hbm.at[pl.ds((base + blk * _CB) * _L, _CB * _L)],
            idx_v.at[islot], sem_i)

    def wait_idx():
        pltpu.make_async_copy(tokens_hbm.at[pl.ds(base * _L, _CB * _L)],
                              idx_v.at[0], sem_i).wait()

    def fire_gath(islot, rib, gs):
        g, sem = bufs[gs]
        pltpu.async_copy(
            table_hbm.at[idx_v.at[islot, pl.ds(rib * _L, _L0)]],
            g.at[pl.ds(0, _L0)], sem)
        pltpu.async_copy(
            table_hbm.at[idx_v.at[islot, pl.ds(rib * _L + _L0, _L1)]],
            g.at[pl.ds(_L0, _L1)], sem)

    def wait_gath(gs):
        g, sem = bufs[gs]
        pltpu.make_async_copy(table_hbm.at[idx_v.at[0, pl.ds(0, _L0)]],
                              g.at[pl.ds(0, _L0)], sem).wait()
        pltpu.make_async_copy(table_hbm.at[idx_v.at[0, pl.ds(_L0, _L1)]],
                              g.at[pl.ds(_L0, _L1)], sem).wait()

    def reduce_row(gs, oslot, rib):
        g, _ = bufs[gs]
        mhi = jnp.int32(-65536)

        def red(i, acc):
            w0 = g[i, pl.ds(0, _LANES)]
            w1 = g[i, pl.ds(_LANES, _LANES)]
            e0 = lax.bitcast_convert_type(w0 << 16, jnp.float32)
            o0 = lax.bitcast_convert_type(w0 & mhi, jnp.float32)
            e1 = lax.bitcast_convert_type(w1 << 16, jnp.float32)
            o1 = lax.bitcast_convert_type(w1 & mhi, jnp.float32)
            return (acc[0] + e0, acc[1] + o0, acc[2] + e1, acc[3] + o1)

        z = jnp.zeros((_LANES,), jnp.float32)
        acc = lax.fori_loop(0, _L, red, (z, z, z, z), unroll=8)
        for j in range(4):
            obuf[oslot, pl.ds(rib * _E + j * _LANES, _LANES)] = acc[j] * inv_l

    def fire_out(blk, oslot):
        pltpu.async_copy(
            obuf.at[oslot],
            pooled_hbm.at[pl.ds((base + blk * _CB) * _E, _CB * _E)],
            osems[oslot])

    def wait_out(oslot):
        pltpu.make_async_copy(obuf.at[oslot],
                              pooled_hbm.at[pl.ds(base * _E, _CB * _E)],
                              osems[oslot]).wait()

    pltpu.sync_copy(tokens_hbm.at[pl.ds(base * _L, _CB * _L)], idx_v.at[0])
    for k in range(_R - 1):
        fire_gath(0, k, k)

    @pl.loop(0, _NBLK, step=2)
    def _(blk0):
        for di in range(2):
            blk = blk0 + di
            islot = di
            oslot = di
            nislot = 1 - di
            has_next = blk + 1 < _NBLK

            @pl.when(has_next)
            def _():
                fire_idx(blk + 1, nislot)

            @pl.when(blk >= 2)
            def _():
                wait_out(oslot)

            @pl.loop(0, (_CB - _R) // _R)
            def _(p):
                r = _R * p
                for k in range(_R):
                    fire_gath(islot, r + k + _R - 1, (k + _R - 1) % _R)
                    wait_gath(k)
                    reduce_row(k, oslot, r + k)

            rt = _CB - _R
            fire_gath(islot, _CB - 1, _R - 1)
            wait_gath(0)
            reduce_row(0, oslot, rt)

            @pl.when(has_next)
            def _():
                wait_idx()
                fire_gath(nislot, 0, 0)

            wait_gath(1)
            reduce_row(1, oslot, rt + 1)
            for k in range(2, _R):
                @pl.when(has_next)
                def _(k=k):
                    fire_gath(nislot, k - 1, k - 1)

                wait_gath(k)
                reduce_row(k, oslot, rt + k)
            fire_out(blk, oslot)

    wait_out(0)
    wait_out(1)


@jax.jit
def _pool_sc(tokens, table):
    mesh = plsc.VectorSubcoreMesh(core_axis_name="c", subcore_axis_name="s")
    f = pl.kernel(
        _pool_body,
        out_type=jax.ShapeDtypeStruct((_B * _E,), jnp.float32),
        mesh=mesh,
        scratch_types=(
            [pltpu.VMEM((2, _CB * _L), jnp.int32)]
            + [pltpu.VMEM((_L, 32), jnp.int32) for _ in range(_R)]
            + [pltpu.VMEM((2, _CB * _E), jnp.float32)]
            + [pltpu.SemaphoreType.DMA] * (_R + 3)
        ),
        compiler_params=pltpu.CompilerParams(use_tc_tiling_on_sc=False),
    )
    return f(tokens, table)


_BB = 512


def _mlp_body(p_ref, w1_ref, b1_ref, w2_ref, b2_ref, o_ref):
    p = p_ref[...]
    h = lax.dot_general(p, w1_ref[...],
                        (((1,), (1,)), ((), ())),
                        preferred_element_type=jnp.float32)
    h = jnp.maximum(h + b1_ref[...], 0.0)
    o = jnp.sum(h * w2_ref[...], axis=1) + b2_ref[0]
    o_ref[...] = jax.nn.sigmoid(o)


@jax.jit
def _mlp_tc(pooled, W1, b1, W2, b2):
    grid = _B // _BB
    return pl.pallas_call(
        _mlp_body,
        grid=(grid,),
        in_specs=[
            pl.BlockSpec((_BB, _E), lambda i: (i, 0)),
            pl.BlockSpec((_H, _E), lambda i: (0, 0)),
            pl.BlockSpec((_H,), lambda i: (0,)),
            pl.BlockSpec((1, _H), lambda i: (0, 0)),
            pl.BlockSpec((1,), lambda i: (0,)),
        ],
        out_specs=pl.BlockSpec((_BB,), lambda i: (i,)),
        out_shape=jax.ShapeDtypeStruct((_B,), jnp.float32),
    )(pooled, W1, b1, W2, b2)


_PERM = (tuple(range(0, 16)) + tuple(range(32, 48))
         + tuple(range(16, 32)) + tuple(range(48, 64)))


def kernel(structure_tokens, table, W1, b1, W2, b2):
    tokens = structure_tokens.astype(jnp.int32).reshape(_B * _L)
    t32 = lax.bitcast_convert_type(table, jnp.int32)
    lo = lax.shift_right_logical(t32[:, :32] + jnp.int32(0x8000), 16)
    hi = (t32[:, 32:] + jnp.int32(0x8000)) & jnp.int32(-65536)
    tbl = hi | lo
    pooled = _pool_sc(tokens, tbl).reshape(_B, _E)
    return _mlp_tc(pooled, W1[:, jnp.array(_PERM)], b1, W2, b2)
